# Initial kernel scaffold; baseline (speedup 1.0000x reference)
import jax
import jax.numpy as jnp
from jax.experimental import pallas as pl


def kernel(noised_image, cover_image, idx_H, idx_W):
    raise NotImplementedError("write your pallas kernel here")


# speedup vs baseline: 3.8162x; 3.8162x over previous
import jax
import jax.numpy as jnp
from jax.experimental import pallas as pl


def _mask_mul_kernel(idx_h_ref, idx_w_ref, x_ref, o_ref):
    _, h, w = x_ref.shape
    n_h = idx_h_ref.shape[1]
    n_w = idx_w_ref.shape[0]
    pos_h = jax.lax.broadcasted_iota(jnp.int32, (h, n_h), 0)
    hit_h = jnp.any(pos_h == idx_h_ref[...], axis=1, keepdims=True)
    pos_w = jax.lax.broadcasted_iota(jnp.int32, (n_w, w), 1)
    hit_w = jnp.any(pos_w == idx_w_ref[...], axis=0, keepdims=True)
    keep_h = jnp.where(hit_h, 0.0, 1.0).astype(x_ref.dtype)
    keep_w = jnp.where(hit_w, 0.0, 1.0).astype(x_ref.dtype)
    mask = keep_h * keep_w
    o_ref[...] = x_ref[...] * mask[None, :, :]


def kernel(noised_image, cover_image, idx_H, idx_W):
    B, C, H, W = noised_image.shape
    x = noised_image.reshape(B * C, H, W)

    def _pad(idx, bound):
        n = idx.shape[0]
        npad = -n % 8
        idx = idx.astype(jnp.int32)
        return jnp.concatenate(
            [idx, jnp.full((npad,), bound, jnp.int32)]) if npad else idx

    ih = _pad(idx_H, H).reshape(1, -1)
    iw = _pad(idx_W, W).reshape(-1, 1)

    G = 8
    out = pl.pallas_call(
        _mask_mul_kernel,
        grid=(B * C // G,),
        in_specs=[
            pl.BlockSpec(ih.shape, lambda i: (0, 0)),
            pl.BlockSpec(iw.shape, lambda i: (0, 0)),
            pl.BlockSpec((G, H, W), lambda i: (i, 0, 0)),
        ],
        out_specs=pl.BlockSpec((G, H, W), lambda i: (i, 0, 0)),
        out_shape=jax.ShapeDtypeStruct((B * C, H, W), x.dtype),
    )(ih, iw, x)
    return (out.reshape(B, C, H, W), cover_image)


# speedup vs baseline: 3.8416x; 1.0067x over previous
import jax
import jax.numpy as jnp
from jax.experimental import pallas as pl


def _mask_mul_kernel(idx_h_ref, idx_w_ref, x_ref, o_ref):
    _, h, w = x_ref.shape
    n_h = idx_h_ref.shape[1]
    n_w = idx_w_ref.shape[0]
    pos_h = jax.lax.broadcasted_iota(jnp.int32, (h, n_h), 0)
    hit_h = jnp.any(pos_h == idx_h_ref[...], axis=1, keepdims=True)
    pos_w = jax.lax.broadcasted_iota(jnp.int32, (n_w, w), 1)
    hit_w = jnp.any(pos_w == idx_w_ref[...], axis=0, keepdims=True)
    keep_h = jnp.where(hit_h, 0.0, 1.0).astype(x_ref.dtype)
    keep_w = jnp.where(hit_w, 0.0, 1.0).astype(x_ref.dtype)
    mask = keep_h * keep_w
    o_ref[...] = x_ref[...] * mask[None, :, :]


def kernel(noised_image, cover_image, idx_H, idx_W):
    B, C, H, W = noised_image.shape
    x = noised_image.reshape(B * C, H, W)

    def _pad(idx, bound):
        n = idx.shape[0]
        npad = -n % 8
        idx = idx.astype(jnp.int32)
        return jnp.concatenate(
            [idx, jnp.full((npad,), bound, jnp.int32)]) if npad else idx

    ih = _pad(idx_H, H).reshape(1, -1)
    iw = _pad(idx_W, W).reshape(-1, 1)

    G = 12
    out = pl.pallas_call(
        _mask_mul_kernel,
        grid=(B * C // G,),
        in_specs=[
            pl.BlockSpec(ih.shape, lambda i: (0, 0)),
            pl.BlockSpec(iw.shape, lambda i: (0, 0)),
            pl.BlockSpec((G, H, W), lambda i: (i, 0, 0)),
        ],
        out_specs=pl.BlockSpec((G, H, W), lambda i: (i, 0, 0)),
        out_shape=jax.ShapeDtypeStruct((B * C, H, W), x.dtype),
    )(ih, iw, x)
    return (out.reshape(B, C, H, W), cover_image)
